# baseline (speedup 1.0000x reference)
import functools

import jax
import jax.numpy as jnp
from jax import lax
from jax.experimental import pallas as pl
from jax.experimental.pallas import tpu as pltpu
from jax.experimental.pallas import tpu_sc as plsc

NC = 2
NS = 16
NW = NC * NS
LANES = 16
CHUNK = 256
GIDX = 128


def _logsigmoid(z):
    u = jnp.exp(-jnp.abs(z))
    s = u / (u + 2.0)
    s2 = s * s
    p = jnp.float32(1.0 / 13.0)
    for c in (1.0 / 11.0, 1.0 / 9.0, 1.0 / 7.0, 1.0 / 5.0, 1.0 / 3.0, 1.0):
        p = p * s2 + jnp.float32(c)
    log1p = 2.0 * s * p
    return jnp.minimum(z, 0.0) - log1p


def _make_sc_kernel(n_tokens, embed):
    per_worker = n_tokens // NW
    n_chunks = per_worker // CHUNK
    hrows_chunk = CHUNK // 2
    hrows_worker = per_worker // 2
    mesh = plsc.VectorSubcoreMesh(core_axis_name="c", subcore_axis_name="s")

    @functools.partial(
        pl.kernel,
        out_type=jax.ShapeDtypeStruct((NW, LANES), jnp.float32),
        mesh=mesh,
        compiler_params=pltpu.CompilerParams(needs_layout_passes=False,
                                             use_tc_tiling_on_sc=False),
        scratch_types=[
            pltpu.VMEM((per_worker,), jnp.int32),
            pltpu.VMEM((hrows_chunk, 2 * embed), jnp.float32),
            pltpu.VMEM((hrows_chunk, 2 * embed), jnp.float32),
            pltpu.VMEM((CHUNK, embed), jnp.float32),
            pltpu.VMEM((CHUNK, embed), jnp.float32),
            pltpu.VMEM((LANES,), jnp.float32),
            pltpu.SemaphoreType.DMA,
            pltpu.SemaphoreType.DMA,
            pltpu.SemaphoreType.DMA,
            pltpu.SemaphoreType.DMA,
        ],
    )
    def sc_kernel(sent_hbm, ctx_hbm, w_hbm, out_hbm, idx_v, ctx_a, ctx_b,
                  wr_a, wr_b, acc_v, sem_ca, sem_cb, sem_wa, sem_wb):
        wid = lax.axis_index("s") * NC + lax.axis_index("c")
        tok0 = wid * per_worker
        hrow0 = wid * hrows_worker
        ctx_bufs = (ctx_a, ctx_b)
        wr_bufs = (wr_a, wr_b)
        ctx_sems = (sem_ca, sem_cb)
        wr_sems = (sem_wa, sem_wb)

        acc_v[...] = jnp.zeros((LANES,), jnp.float32)
        pltpu.sync_copy(sent_hbm.at[pl.ds(tok0, per_worker)], idx_v)

        def launch(k, b):
            pltpu.async_copy(
                ctx_hbm.at[pl.ds(hrow0 + k * hrows_chunk, hrows_chunk)],
                ctx_bufs[b], ctx_sems[b])
            for j in range(CHUNK // GIDX):
                pltpu.async_copy(
                    w_hbm.at[idx_v.at[pl.ds(k * CHUNK + j * GIDX, GIDX)]],
                    wr_bufs[b].at[pl.ds(j * GIDX, GIDX)], wr_sems[b])

        def wait(k, b):
            pltpu.make_async_copy(
                ctx_hbm.at[pl.ds(hrow0 + k * hrows_chunk, hrows_chunk)],
                ctx_bufs[b], ctx_sems[b]).wait()
            for j in range(CHUNK // GIDX):
                pltpu.make_async_copy(
                    w_hbm.at[idx_v.at[pl.ds(k * CHUNK + j * GIDX, GIDX)]],
                    wr_bufs[b].at[pl.ds(j * GIDX, GIDX)], wr_sems[b]).wait()

        def compute(b):
            ctx_r = ctx_bufs[b]
            wr_r = wr_bufs[b]

            def group_body(g, carry):
                rows = g * LANES + lax.iota(jnp.int32, LANES)
                wre = rows * 2
                wro = wre + 1
                ze0 = jnp.zeros((LANES,), jnp.float32)
                ze1 = jnp.zeros((LANES,), jnp.float32)
                zo0 = jnp.zeros((LANES,), jnp.float32)
                zo1 = jnp.zeros((LANES,), jnp.float32)
                for e in range(embed):
                    ev = jnp.full((LANES,), e, jnp.int32)
                    ev2 = jnp.full((LANES,), e + embed, jnp.int32)
                    we = plsc.load_gather(wr_r, [wre, ev])
                    ce = plsc.load_gather(ctx_r, [rows, ev])
                    wo = plsc.load_gather(wr_r, [wro, ev])
                    co = plsc.load_gather(ctx_r, [rows, ev2])
                    if e % 2 == 0:
                        ze0 = ze0 + we * ce
                        zo0 = zo0 + wo * co
                    else:
                        ze1 = ze1 + we * ce
                        zo1 = zo1 + wo * co
                acc_v[...] = (acc_v[...] + _logsigmoid(ze0 + ze1)
                              + _logsigmoid(zo0 + zo1))
                return carry

            lax.fori_loop(0, hrows_chunk // LANES, group_body, 0)

        launch(0, 0)

        def pair_body(k2, carry):
            k = 2 * k2
            launch(k + 1, 1)
            wait(k, 0)
            compute(0)

            @pl.when(k + 2 < n_chunks)
            def _():
                launch(k + 2, 0)

            wait(k + 1, 1)
            compute(1)
            return carry

        lax.fori_loop(0, n_chunks // 2, pair_body, 0)
        pltpu.sync_copy(acc_v, out_hbm.at[wid])

    return sc_kernel


def kernel(sentence, context, W):
    b, l = sentence.shape
    embed = W.shape[1]
    n_tokens = b * l
    sent_flat = sentence.reshape(n_tokens)
    ctx_half = context.reshape(n_tokens // 2, 2 * embed)
    partials = _make_sc_kernel(n_tokens, embed)(sent_flat, ctx_half, W)
    return (-jnp.sum(partials) / b).astype(jnp.float32)


# speedup vs baseline: 1.1184x; 1.1184x over previous
import functools

import jax
import jax.numpy as jnp
from jax import lax
from jax.experimental import pallas as pl
from jax.experimental.pallas import tpu as pltpu
from jax.experimental.pallas import tpu_sc as plsc

NC = 2
NS = 16
NW = NC * NS
LANES = 16
CHUNK = 256
GIDX = 128


def _logsigmoid(z):
    u = jnp.exp(-jnp.abs(z))
    s = u / (u + 2.0)
    s2 = s * s
    p = jnp.float32(1.0 / 13.0)
    for c in (1.0 / 11.0, 1.0 / 9.0, 1.0 / 7.0, 1.0 / 5.0, 1.0 / 3.0, 1.0):
        p = p * s2 + jnp.float32(c)
    log1p = 2.0 * s * p
    return jnp.minimum(z, 0.0) - log1p


def _make_sc_kernel(n_tokens, embed):
    per_worker = n_tokens // NW
    n_chunks = per_worker // CHUNK
    hrows_chunk = CHUNK // 2
    hrows_worker = per_worker // 2
    mesh = plsc.VectorSubcoreMesh(core_axis_name="c", subcore_axis_name="s")

    @functools.partial(
        pl.kernel,
        out_type=jax.ShapeDtypeStruct((NW, LANES), jnp.float32),
        mesh=mesh,
        compiler_params=pltpu.CompilerParams(needs_layout_passes=False,
                                             use_tc_tiling_on_sc=False),
        scratch_types=[
            pltpu.VMEM((per_worker,), jnp.int32),
            pltpu.VMEM((hrows_chunk, 2 * embed), jnp.float32),
            pltpu.VMEM((hrows_chunk, 2 * embed), jnp.float32),
            pltpu.VMEM((CHUNK, embed), jnp.float32),
            pltpu.VMEM((CHUNK, embed), jnp.float32),
            pltpu.VMEM((LANES,), jnp.float32),
            pltpu.SemaphoreType.DMA,
            pltpu.SemaphoreType.DMA,
            pltpu.SemaphoreType.DMA,
            pltpu.SemaphoreType.DMA,
        ],
    )
    def sc_kernel(sent_hbm, ctx_hbm, w_hbm, out_hbm, idx_v, ctx_a, ctx_b,
                  wr_a, wr_b, acc_v, sem_ca, sem_cb, sem_wa, sem_wb):
        wid = lax.axis_index("s") * NC + lax.axis_index("c")
        tok0 = wid * per_worker
        hrow0 = wid * hrows_worker
        ctx_bufs = (ctx_a, ctx_b)
        wr_bufs = (wr_a, wr_b)
        ctx_sems = (sem_ca, sem_cb)
        wr_sems = (sem_wa, sem_wb)

        acc_v[...] = jnp.zeros((LANES,), jnp.float32)
        pltpu.sync_copy(sent_hbm.at[pl.ds(tok0, per_worker)], idx_v)

        def launch(k, b):
            pltpu.async_copy(
                ctx_hbm.at[pl.ds(hrow0 + k * hrows_chunk, hrows_chunk)],
                ctx_bufs[b], ctx_sems[b])
            for j in range(CHUNK // GIDX):
                pltpu.async_copy(
                    w_hbm.at[idx_v.at[pl.ds(k * CHUNK + j * GIDX, GIDX)]],
                    wr_bufs[b].at[pl.ds(j * GIDX, GIDX)], wr_sems[b])

        def wait(k, b):
            pltpu.make_async_copy(
                ctx_hbm.at[pl.ds(hrow0 + k * hrows_chunk, hrows_chunk)],
                ctx_bufs[b], ctx_sems[b]).wait()
            for j in range(CHUNK // GIDX):
                pltpu.make_async_copy(
                    w_hbm.at[idx_v.at[pl.ds(k * CHUNK + j * GIDX, GIDX)]],
                    wr_bufs[b].at[pl.ds(j * GIDX, GIDX)], wr_sems[b]).wait()

        def compute(b):
            ctx_r = ctx_bufs[b]
            wr_r = wr_bufs[b]

            def group_body(g, carry):
                rows = g * LANES + lax.iota(jnp.int32, LANES)
                wre = rows * 2
                wro = wre + 1
                zinit = jnp.zeros((LANES,), jnp.float32)

                def dot_step(e, z):
                    ze0, ze1, zo0, zo1 = z
                    ev = jnp.full((LANES,), 1, jnp.int32) * e
                    ev1 = ev + 1
                    ze0 = ze0 + (plsc.load_gather(wr_r, [wre, ev])
                                 * plsc.load_gather(ctx_r, [rows, ev]))
                    zo0 = zo0 + (plsc.load_gather(wr_r, [wro, ev])
                                 * plsc.load_gather(ctx_r, [rows, ev + embed]))
                    ze1 = ze1 + (plsc.load_gather(wr_r, [wre, ev1])
                                 * plsc.load_gather(ctx_r, [rows, ev1]))
                    zo1 = zo1 + (plsc.load_gather(wr_r, [wro, ev1])
                                 * plsc.load_gather(ctx_r, [rows, ev1 + embed]))
                    return ze0, ze1, zo0, zo1

                ze0, ze1, zo0, zo1 = plsc.parallel_loop(
                    0, embed, step=2, unroll=4,
                    carry=(zinit, zinit, zinit, zinit))(dot_step)
                acc_v[...] = (acc_v[...] + _logsigmoid(ze0 + ze1)
                              + _logsigmoid(zo0 + zo1))
                return carry

            lax.fori_loop(0, hrows_chunk // LANES, group_body, 0)

        launch(0, 0)

        def pair_body(k2, carry):
            k = 2 * k2
            launch(k + 1, 1)
            wait(k, 0)
            compute(0)

            @pl.when(k + 2 < n_chunks)
            def _():
                launch(k + 2, 0)

            wait(k + 1, 1)
            compute(1)
            return carry

        lax.fori_loop(0, n_chunks // 2, pair_body, 0)
        pltpu.sync_copy(acc_v, out_hbm.at[wid])

    return sc_kernel


def kernel(sentence, context, W):
    b, l = sentence.shape
    embed = W.shape[1]
    n_tokens = b * l
    sent_flat = sentence.reshape(n_tokens)
    ctx_half = context.reshape(n_tokens // 2, 2 * embed)
    partials = _make_sc_kernel(n_tokens, embed)(sent_flat, ctx_half, W)
    return (-jnp.sum(partials) / b).astype(jnp.float32)


# speedup vs baseline: 1.2588x; 1.1256x over previous
import functools

import jax
import jax.numpy as jnp
from jax import lax
from jax.experimental import pallas as pl
from jax.experimental.pallas import tpu as pltpu
from jax.experimental.pallas import tpu_sc as plsc

NC = 2
NS = 16
NW = NC * NS
LANES = 16
CHUNK = 128
PADE = 128


def _logsigmoid(z):
    u = jnp.exp(-jnp.abs(z))
    s = u / (u + 2.0)
    s2 = s * s
    p = jnp.float32(1.0 / 13.0)
    for c in (1.0 / 11.0, 1.0 / 9.0, 1.0 / 7.0, 1.0 / 5.0, 1.0 / 3.0, 1.0):
        p = p * s2 + jnp.float32(c)
    return jnp.minimum(z, 0.0) - 2.0 * s * p


def _make_sc_kernel(n_tokens, embed):
    per_worker = n_tokens // NW
    n_chunks = per_worker // CHUNK
    mesh = plsc.VectorSubcoreMesh(core_axis_name="c", subcore_axis_name="s")

    @functools.partial(
        pl.kernel,
        out_type=jax.ShapeDtypeStruct((NW, LANES), jnp.float32),
        mesh=mesh,
        compiler_params=pltpu.CompilerParams(needs_layout_passes=False),
        scratch_types=[
            pltpu.VMEM((per_worker,), jnp.int32),
            pltpu.VMEM((CHUNK, embed), jnp.float32),
            pltpu.VMEM((CHUNK, embed), jnp.float32),
            pltpu.VMEM((CHUNK, PADE), jnp.float32),
            pltpu.VMEM((CHUNK, PADE), jnp.float32),
            pltpu.VMEM((LANES,), jnp.float32),
            pltpu.SemaphoreType.DMA,
            pltpu.SemaphoreType.DMA,
            pltpu.SemaphoreType.DMA,
            pltpu.SemaphoreType.DMA,
        ],
    )
    def sc_kernel(sent_hbm, ctx_hbm, w_hbm, out_hbm, idx_v, ctx_a, ctx_b,
                  wr_a, wr_b, acc_v, sem_ca, sem_cb, sem_wa, sem_wb):
        wid = lax.axis_index("s") * NC + lax.axis_index("c")
        tok0 = wid * per_worker
        ctx_bufs = (ctx_a, ctx_b)
        wr_bufs = (wr_a, wr_b)
        ctx_sems = (sem_ca, sem_cb)
        wr_sems = (sem_wa, sem_wb)

        acc_v[...] = jnp.zeros((LANES,), jnp.float32)
        pltpu.sync_copy(sent_hbm.at[pl.ds(tok0, per_worker)], idx_v)

        def launch(k, b):
            pltpu.async_copy(
                ctx_hbm.at[pl.ds(tok0 + k * CHUNK, CHUNK)],
                ctx_bufs[b], ctx_sems[b])
            pltpu.async_copy(
                w_hbm.at[idx_v.at[pl.ds(k * CHUNK, CHUNK)]],
                wr_bufs[b], wr_sems[b])

        def wait(k, b):
            pltpu.make_async_copy(
                ctx_hbm.at[pl.ds(tok0 + k * CHUNK, CHUNK)],
                ctx_bufs[b], ctx_sems[b]).wait()
            pltpu.make_async_copy(
                w_hbm.at[idx_v.at[pl.ds(k * CHUNK, CHUNK)]],
                wr_bufs[b], wr_sems[b]).wait()

        def compute(b):
            ctx_r = ctx_bufs[b]
            wr_r = wr_bufs[b]

            def group_body(g, carry):
                rows = g * LANES + lax.iota(jnp.int32, LANES)
                zinit = jnp.zeros((LANES,), jnp.float32)

                def dot_step(e, z):
                    z0, z1 = z
                    ev = jnp.zeros((LANES,), jnp.int32) + e
                    ev1 = ev + 1
                    z0 = z0 + (plsc.load_gather(wr_r, [rows, ev])
                               * plsc.load_gather(ctx_r, [rows, ev]))
                    z1 = z1 + (plsc.load_gather(wr_r, [rows, ev1])
                               * plsc.load_gather(ctx_r, [rows, ev1]))
                    return z0, z1

                z0, z1 = plsc.parallel_loop(
                    0, embed, step=2, unroll=4,
                    carry=(zinit, zinit))(dot_step)
                acc_v[...] = acc_v[...] + _logsigmoid(z0 + z1)
                return carry

            lax.fori_loop(0, CHUNK // LANES, group_body, 0)

        launch(0, 0)

        def pair_body(k2, carry):
            k = 2 * k2
            launch(k + 1, 1)
            wait(k, 0)
            compute(0)

            @pl.when(k + 2 < n_chunks)
            def _():
                launch(k + 2, 0)

            wait(k + 1, 1)
            compute(1)
            return carry

        lax.fori_loop(0, n_chunks // 2, pair_body, 0)
        pltpu.sync_copy(acc_v, out_hbm.at[wid])

    return sc_kernel


def kernel(sentence, context, W):
    b, l = sentence.shape
    embed = W.shape[1]
    n_tokens = b * l
    sent_flat = sentence.reshape(n_tokens)
    ctx_flat = context.reshape(n_tokens, embed)
    w_pad = jnp.pad(W, ((0, 0), (0, PADE - embed)))
    partials = _make_sc_kernel(n_tokens, embed)(sent_flat, ctx_flat, w_pad)
    return (-jnp.sum(partials) / b).astype(jnp.float32)
